# baseline (speedup 1.0000x reference)
import jax
import jax.numpy as jnp
from jax import lax
from jax.experimental import pallas as pl
from jax.experimental.pallas import tpu as pltpu
from jax.experimental.pallas import tpu_sc as plsc

VOCAB = 100000
EMB = 64
CTX = 20
HID = 128
NIDX = 2 * CTX
FLAT = NIDX * EMB

IDX_PAD = 48

BC = 8192
NBA = 7
C0 = NBA * BC
C0CT = C0 // 128
LAST_CT = (VOCAB - 1) // 128
Q = 11
CBP = 32 * Q * 128
CBV = VOCAB - C0


def _sc_gather_body(table_hbm, idx_hbm, out_hbm, idx_v, rows_v, sem):
    wid = lax.axis_index("s") * 2 + lax.axis_index("c")

    @pl.when(wid == 0)
    def _():
        pltpu.sync_copy(idx_hbm, idx_v.at[pl.ds(0, NIDX)])
        copies = []
        for i in range(NIDX):
            c, l = divmod(i, 16)
            v = idx_v[pl.ds(c * 16, 16)]
            s = v[l]
            s = jnp.minimum(jnp.maximum(s, 0), VOCAB - 1)
            copies.append(pltpu.async_copy(
                table_hbm.at[pl.ds(s, 1)], rows_v.at[pl.ds(i, 1)], sem))
        for cp in copies:
            cp.wait()
        pltpu.sync_copy(rows_v, out_hbm)


def _sc_gather(table, idx):
    mesh = plsc.VectorSubcoreMesh(core_axis_name="c", subcore_axis_name="s")
    k = pl.kernel(
        _sc_gather_body,
        out_type=jax.ShapeDtypeStruct((NIDX, EMB), jnp.float32),
        mesh=mesh,
        scratch_types=[
            pltpu.VMEM((IDX_PAD,), jnp.int32),
            pltpu.VMEM((NIDX, EMB), jnp.float32),
            pltpu.SemaphoreType.DMA,
        ],
    )
    return k(table, idx)


def _fc1_body(emb_ref, w1_ref, b1_ref, out_ref):
    h = b1_ref[...]
    for i in range(NIDX):
        h = h + jnp.dot(emb_ref[pl.ds(i, 1), :], w1_ref[i],
                        preferred_element_type=jnp.float32)
    out_ref[...] = jnp.maximum(h, 0.0)


def _tc_fc1(emb, W1r, b1):
    return pl.pallas_call(
        _fc1_body,
        out_shape=jax.ShapeDtypeStruct((1, HID), jnp.float32),
    )(emb, W1r, b1)


def _sc_mv_body(hid_hbm, w2_hbm, out_hbm, hid_v, buf_v, log_v, sem):
    c = lax.axis_index("c")
    s = lax.axis_index("s")
    slot0 = (c * 16 + s) * Q

    pltpu.sync_copy(hid_hbm.at[0], hid_v)
    hs = []
    for ch in range(8):
        v = hid_v[pl.ds(ch * 16, 16)]
        for l in range(16):
            hs.append(v[l])

    def g_body(g, carry):
        ct = C0CT + slot0 + g

        @pl.when(ct <= LAST_CT)
        def _():
            lane = pl.multiple_of(ct * 128, 128)
            copies = [
                pltpu.async_copy(
                    w2_hbm.at[pl.ds(rt * 8, 8), pl.ds(lane, 128)],
                    buf_v.at[rt], sem)
                for rt in range(16)
            ]
            for cp in copies:
                cp.wait()
            zero = jnp.zeros((16,), jnp.float32)
            acc = [zero for _ in range(8)]
            for rt in range(16):
                for sr in range(8):
                    hk = hs[rt * 8 + sr]
                    for vv in range(8):
                        acc[vv] = acc[vv] + hk * buf_v[rt, sr,
                                                       pl.ds(vv * 16, 16)]
            for vv in range(8):
                log_v[pl.ds(vv * 16, 16)] = acc[vv]
            off = pl.multiple_of((slot0 + g) * 128, 128)
            pltpu.sync_copy(log_v, out_hbm.at[pl.ds(off, 128)])
        return carry

    lax.fori_loop(0, Q, g_body, 0)


def _sc_matvec(hid, W2):
    mesh = plsc.VectorSubcoreMesh(core_axis_name="c", subcore_axis_name="s")
    k = pl.kernel(
        _sc_mv_body,
        out_type=jax.ShapeDtypeStruct((CBP,), jnp.float32),
        mesh=mesh,
        scratch_types=[
            pltpu.VMEM((HID,), jnp.float32),
            pltpu.VMEM((16, 8, 128), jnp.float32),
            pltpu.VMEM((128,), jnp.float32),
            pltpu.SemaphoreType.DMA,
        ],
    )
    return k(hid, W2)


def _tc_stream_body(hid_ref, w2_ref, b2_ref, out_ref):
    j = pl.program_id(0)
    blk = jnp.dot(hid_ref[...], w2_ref[...], preferred_element_type=jnp.float32)
    off = pl.multiple_of(j * BC, BC)
    out_ref[:, pl.ds(off, BC)] = blk + b2_ref[...]


def _tc_stream(hid, W2, b2):
    return pl.pallas_call(
        _tc_stream_body,
        grid=(NBA,),
        in_specs=[
            pl.BlockSpec((1, HID), lambda j: (0, 0)),
            pl.BlockSpec((HID, BC), lambda j: (0, j)),
            pl.BlockSpec((1, BC), lambda j: (0, j)),
        ],
        out_specs=pl.BlockSpec((1, C0), lambda j: (0, 0)),
        out_shape=jax.ShapeDtypeStruct((1, C0), jnp.float32),
    )(hid, W2, b2)


def _softmax_body(la_ref, lb_ref, b2b_ref, out_ref):
    a = la_ref[...]
    bb = lb_ref[...][:, :CBV] + b2b_ref[...]
    m = jnp.maximum(jnp.max(a), jnp.max(bb))
    ssum = jnp.sum(jnp.exp(a - m)) + jnp.sum(jnp.exp(bb - m))
    cst = m + jnp.log(ssum)
    out_ref[:, :C0] = a - cst
    out_ref[:, C0:] = bb - cst


def _tc_softmax(la, lb, b2b):
    return pl.pallas_call(
        _softmax_body,
        out_shape=jax.ShapeDtypeStruct((1, VOCAB), jnp.float32),
    )(la, lb, b2b)


def kernel(inputs, table, W1, b1, W2, b2):
    emb = _sc_gather(table, inputs)
    W1r = W1.reshape(NIDX, EMB, HID)
    hid = _tc_fc1(emb, W1r, b1.reshape(1, HID))
    lb = _sc_matvec(hid, W2)
    la = _tc_stream(hid, W2, b2.reshape(1, VOCAB))
    return _tc_softmax(la, lb.reshape(1, CBP), b2[C0:].reshape(1, CBV))


# speedup vs baseline: 1.3065x; 1.3065x over previous
import jax
import jax.numpy as jnp
from jax import lax
from jax.experimental import pallas as pl
from jax.experimental.pallas import tpu as pltpu
from jax.experimental.pallas import tpu_sc as plsc

VOCAB = 100000
EMB = 64
CTX = 20
HID = 128
NIDX = 2 * CTX
FLAT = NIDX * EMB

BC = 16384
NB = -(-VOCAB // BC)
PADV = NB * BC

IDX_PAD = 48


def _sc_gather_body(table_hbm, idx_hbm, out_hbm, idx_v, rows_v, sem):
    wid = lax.axis_index("s") * 2 + lax.axis_index("c")

    @pl.when(wid == 0)
    def _():
        pltpu.sync_copy(idx_hbm, idx_v.at[pl.ds(0, NIDX)])
        copies = []
        for i in range(NIDX):
            c, l = divmod(i, 16)
            v = idx_v[pl.ds(c * 16, 16)]
            s = v[l]
            s = jnp.minimum(jnp.maximum(s, 0), VOCAB - 1)
            copies.append(pltpu.async_copy(
                table_hbm.at[pl.ds(s, 1)], rows_v.at[pl.ds(i, 1)], sem))
        for cp in copies:
            cp.wait()
        pltpu.sync_copy(rows_v, out_hbm)


def _sc_gather(table, idx):
    mesh = plsc.VectorSubcoreMesh(core_axis_name="c", subcore_axis_name="s")
    k = pl.kernel(
        _sc_gather_body,
        out_type=jax.ShapeDtypeStruct((NIDX, EMB), jnp.float32),
        mesh=mesh,
        scratch_types=[
            pltpu.VMEM((IDX_PAD,), jnp.int32),
            pltpu.VMEM((NIDX, EMB), jnp.float32),
            pltpu.SemaphoreType.DMA,
        ],
    )
    return k(table, idx)


def _tc_body(emb_ref, w1_ref, b1_ref, w2_ref, b2_ref, out_ref,
             hid_ref, rmax_ref):
    j = pl.program_id(0)

    @pl.when(j == 0)
    def _init():
        h = b1_ref[...]
        for i in range(NIDX):
            h = h + jnp.dot(emb_ref[pl.ds(i, 1), :], w1_ref[i],
                            preferred_element_type=jnp.float32)
        hid_ref[...] = jnp.maximum(h, 0.0)
        rmax_ref[...] = jnp.full((1, BC), -jnp.inf, jnp.float32)

    blk = jnp.dot(hid_ref[...], w2_ref[...], preferred_element_type=jnp.float32)
    blk = blk + b2_ref[...]
    off = pl.multiple_of(j * BC, BC)
    out_ref[:, pl.ds(off, BC)] = blk

    @pl.when(j < NB - 1)
    def _run():
        rmax_ref[...] = jnp.maximum(rmax_ref[...], blk)

    @pl.when(j == NB - 1)
    def _fin():
        iota = lax.broadcasted_iota(jnp.int32, (1, BC), 1)
        lastvalid = (j * BC + iota) < VOCAB
        rmax = jnp.maximum(rmax_ref[...],
                           jnp.where(lastvalid, blk, -jnp.inf))
        m = jnp.max(rmax)
        row = out_ref[...]
        cols = lax.broadcasted_iota(jnp.int32, (1, PADV), 1)
        valid = cols < VOCAB
        ssum = jnp.sum(jnp.where(valid, jnp.exp(row - m), 0.0))
        out_ref[...] = row - (m + jnp.log(ssum))


def _tc_mlp(emb, W1r, b1, W2, b2):
    out = pl.pallas_call(
        _tc_body,
        grid=(NB,),
        in_specs=[
            pl.BlockSpec((NIDX, EMB), lambda j: (0, 0)),
            pl.BlockSpec((NIDX, EMB, HID), lambda j: (0, 0, 0)),
            pl.BlockSpec((1, HID), lambda j: (0, 0)),
            pl.BlockSpec((HID, BC), lambda j: (0, j)),
            pl.BlockSpec((1, BC), lambda j: (0, j)),
        ],
        out_specs=pl.BlockSpec((1, PADV), lambda j: (0, 0)),
        out_shape=jax.ShapeDtypeStruct((1, PADV), jnp.float32),
        scratch_shapes=[
            pltpu.VMEM((1, HID), jnp.float32),
            pltpu.VMEM((1, BC), jnp.float32),
        ],
    )(emb, W1r, b1, W2, b2)
    return out[:, :VOCAB]


def kernel(inputs, table, W1, b1, W2, b2):
    emb = _sc_gather(table, inputs)
    W1r = W1.reshape(NIDX, EMB, HID)
    return _tc_mlp(emb, W1r, b1.reshape(1, HID), W2, b2.reshape(1, VOCAB))
